# baseline (speedup 1.0000x reference)
import math

import jax
import jax.numpy as jnp
from jax import lax
from jax.experimental import pallas as pl
from jax.experimental.pallas import tpu as pltpu
from jax.experimental.pallas import tpu_sc as plsc

_BASE = 2
_REP = 8
_SREP = 4
_SPMEM_LEVELS = 2


def _sc_body(emb_hbm, out_hbm, pat, idx, spat, gsem, sem, ssem):
    cid = lax.axis_index("c")
    sid = lax.axis_index("s")
    wid = sid * 2 + cid
    max_level, t_total, _ = out_hbm.shape
    chunk = t_total // 32
    t0 = wid * chunk
    split = max_level - _SPMEM_LEVELS
    npad = idx.shape[0]

    for c0 in range(0, npad, 16):
        j = c0 + lax.iota(jnp.int32, 16)
        lvl = j >> 3
        par = j & 1
        vals = (lvl << 1) + par * (1 - (lvl & 1))
        vals = jnp.where(lvl < max_level, vals, 0)
        idx[pl.ds(c0, 16)] = vals
    pltpu.async_copy(emb_hbm.at[idx], pat, gsem).wait()

    for i in range(_SPMEM_LEVELS):
        pltpu.sync_copy(pat.at[pl.ds((split + i) * _REP, _SREP)],
                        spat.at[sid, pl.ds(i * _SREP, _SREP)])

    pending = []
    for i in range(_SPMEM_LEVELS):
        l = split + i
        src = spat.at[sid, pl.ds(i * _SREP, _SREP)]
        for k in range(chunk // _SREP):
            pending.append(pltpu.async_copy(
                src, out_hbm.at[l, pl.ds(t0 + k * _SREP, _SREP)], ssem))
    for l in range(split):
        src = pat.at[pl.ds(l * _REP, _REP)]
        for k in range(chunk // _REP):
            pending.append(pltpu.async_copy(
                src, out_hbm.at[l, pl.ds(t0 + k * _REP, _REP)], sem))
    for h in pending:
        h.wait()


def kernel(x, emb):
    B, T = x.shape
    del B
    max_level = int(math.ceil(math.log(T, _BASE)))
    d = emb.shape[1]
    npad = -(-max_level * _REP // 16) * 16

    mesh = plsc.VectorSubcoreMesh(core_axis_name="c", subcore_axis_name="s")
    k = pl.kernel(
        _sc_body,
        out_type=jax.ShapeDtypeStruct((max_level, T, d), emb.dtype),
        mesh=mesh,
        scratch_types=[
            pltpu.VMEM((npad, d), emb.dtype),
            pltpu.VMEM((npad,), jnp.int32),
            pltpu.VMEM_SHARED((16, _SPMEM_LEVELS * _SREP, d), emb.dtype),
            pltpu.SemaphoreType.DMA,
            pltpu.SemaphoreType.DMA,
            pltpu.SemaphoreType.DMA,
        ],
    )
    return k(emb)


# speedup vs baseline: 1.0179x; 1.0179x over previous
import math

import jax
import jax.numpy as jnp
from jax import lax
from jax.experimental import pallas as pl
from jax.experimental.pallas import tpu as pltpu
from jax.experimental.pallas import tpu_sc as plsc

_BASE = 2
_REP = 8


def _sc_body(emb_hbm, out_hbm, pat, idx, gsem, sem):
    cid = lax.axis_index("c")
    sid = lax.axis_index("s")
    wid = sid * 2 + cid
    max_level, t_total, _ = out_hbm.shape
    chunk = t_total // 32
    t0 = wid * chunk
    nstream = chunk // _REP
    npad = idx.shape[0]

    for c0 in range(0, npad, 16):
        j = c0 + lax.iota(jnp.int32, 16)
        lvl = j >> 3
        par = j & 1
        vals = (lvl << 1) + par * (1 - (lvl & 1))
        vals = jnp.where(lvl < max_level, vals, 0)
        idx[pl.ds(c0, 16)] = vals
    pltpu.async_copy(emb_hbm.at[idx], pat, gsem).wait()

    pending = []
    for l in range(max_level):
        src = pat.at[pl.ds(l * _REP, _REP)]
        for k in range(nstream):
            h = pltpu.async_copy(
                src, out_hbm.at[l, pl.ds(t0 + k * _REP, _REP)], sem)
            pending.append(h)
    for h in pending:
        h.wait()


def kernel(x, emb):
    B, T = x.shape
    del B
    max_level = int(math.ceil(math.log(T, _BASE)))
    d = emb.shape[1]
    npad = -(-max_level * _REP // 16) * 16

    mesh = plsc.VectorSubcoreMesh(core_axis_name="c", subcore_axis_name="s")
    k = pl.kernel(
        _sc_body,
        out_type=jax.ShapeDtypeStruct((max_level, T, d), emb.dtype),
        mesh=mesh,
        scratch_types=[
            pltpu.VMEM((npad, d), emb.dtype),
            pltpu.VMEM((npad,), jnp.int32),
            pltpu.SemaphoreType.DMA,
            pltpu.SemaphoreType.DMA,
        ],
    )
    return k(emb)
